# Initial kernel scaffold; baseline (speedup 1.0000x reference)
import jax
import jax.numpy as jnp
from jax.experimental import pallas as pl


def kernel(feats, labels, cam_ids):
    raise NotImplementedError("write your pallas kernel here")


# speedup vs baseline: 3.5082x; 3.5082x over previous
import functools

import jax
import jax.numpy as jnp
from jax import lax
from jax.experimental import pallas as pl
from jax.experimental.pallas import tpu as pltpu
from jax.experimental.pallas import tpu_sc as plsc

N_LABELS = 1000
N_CAMS = 8
B = 16384
D = 128
G = N_LABELS * N_CAMS
GP = 8192
NC = 2
NS = 16
L = 16

ROWS_PER_TILE = B // NS
HALF = B // NC
LROWS = HALF // NS
CHUNK = 128
G_PER_TILE = GP // NS


def _sc_body(feats_hbm, labels_hbm, cams_hbm, out_hbm,
             sums_sh, cnt_sh, labv, camv, idx_v, fbuf, cgbuf, onesv,
             cntv, accv, sem):
    cid = lax.axis_index("c")
    sid = lax.axis_index("s")

    def _z(r, _):
        for c in range(D // L):
            fbuf[r, pl.ds(c * L, L)] = jnp.zeros((L,), jnp.float32)
        return 0

    lax.fori_loop(0, CHUNK, _z, 0)
    for q in range(CHUNK // L):
        onesv[pl.ds(q * L, L)] = jnp.ones((L,), jnp.float32)

    def _z2(r, _):
        cntv[pl.ds(r * L, L)] = jnp.zeros((L,), jnp.float32)
        return 0

    lax.fori_loop(0, G_PER_TILE // L, _z2, 0)
    gbase = sid * G_PER_TILE
    for k in range(G_PER_TILE // CHUNK):
        pltpu.sync_copy(fbuf, sums_sh.at[pl.ds(gbase + k * CHUNK, CHUNK)])
    pltpu.sync_copy(cntv.at[pl.ds(0, G_PER_TILE)],
                    cnt_sh.at[pl.ds(gbase, G_PER_TILE)])

    for h in (0, 1):
        off = ((cid + h) % 2) * HALF + sid * LROWS
        pltpu.sync_copy(labels_hbm.at[pl.ds(off, LROWS)],
                        labv.at[pl.ds(h * LROWS, LROWS)])
        pltpu.sync_copy(cams_hbm.at[pl.ds(off, LROWS)],
                        camv.at[pl.ds(h * LROWS, LROWS)])

    for q in range(ROWS_PER_TILE // L):
        g = labv[pl.ds(q * L, L)] * N_CAMS + camv[pl.ds(q * L, L)]
        idx_v[q // (CHUNK // L), pl.ds((q % (CHUNK // L)) * L, L)] = g

    plsc.subcore_barrier()

    for r in range(ROWS_PER_TILE // CHUNK):
        h = r // (LROWS // CHUNK)
        off = ((cid + h) % 2) * HALF + sid * LROWS + (r % (LROWS // CHUNK)) * CHUNK
        pltpu.sync_copy(feats_hbm.at[pl.ds(off, CHUNK)], fbuf)
        pltpu.sync_copy(fbuf, sums_sh.at[idx_v.at[r]], add=True)
        pltpu.sync_copy(onesv, cnt_sh.at[idx_v.at[r]], add=True)

    plsc.subcore_barrier()

    pltpu.sync_copy(cnt_sh.at[pl.ds(gbase, G_PER_TILE)],
                    cntv.at[pl.ds(0, G_PER_TILE)])
    for k in range(G_PER_TILE // CHUNK):
        roff = gbase + k * CHUNK
        pltpu.sync_copy(sums_sh.at[pl.ds(roff, CHUNK)], fbuf)

        def _norm(r, _):
            cv = 1.0 / jnp.maximum(cntv[pl.ds(k * CHUNK + r, L)], 1.0)
            inv = jnp.full((L,), cv[0], jnp.float32)
            for c in range(D // L):
                fbuf[r, pl.ds(c * L, L)] = fbuf[r, pl.ds(c * L, L)] * inv
            return 0

        lax.fori_loop(0, CHUNK, _norm, 0)
        pltpu.sync_copy(fbuf, sums_sh.at[pl.ds(roff, CHUNK)])

    plsc.subcore_barrier()

    acc = jnp.zeros((L,), jnp.float32)
    for r in range(LROWS // CHUNK):
        off = cid * HALF + sid * LROWS + r * CHUNK
        pltpu.sync_copy(feats_hbm.at[pl.ds(off, CHUNK)], fbuf)
        pltpu.async_copy(sums_sh.at[idx_v.at[r]], cgbuf, sem).wait()

        def _loss(r2, a):
            for c in range(D // L):
                f = fbuf[r2, pl.ds(c * L, L)]
                t = cgbuf[r2, pl.ds(c * L, L)]
                d = f - t
                ad = jnp.abs(d)
                a = a + jnp.where(ad < 1.0, 0.5 * d * d, ad - 0.5)
            return a

        acc = lax.fori_loop(0, CHUNK, _loss, acc)

    accv[pl.ds(0, 1), pl.ds(0, L)] = acc.reshape(1, L)
    wid = cid * NS + sid
    pltpu.sync_copy(accv, out_hbm.at[pl.ds(wid, 1)])


_sc_kernel = functools.partial(
    pl.kernel,
    out_type=jax.ShapeDtypeStruct((NC * NS, L), jnp.float32),
    mesh=plsc.VectorSubcoreMesh(core_axis_name="c", subcore_axis_name="s"),
    scratch_types=[
        pltpu.VMEM_SHARED((GP, D), jnp.float32),
        pltpu.VMEM_SHARED((GP,), jnp.float32),
        pltpu.VMEM((ROWS_PER_TILE,), jnp.int32),
        pltpu.VMEM((ROWS_PER_TILE,), jnp.int32),
        pltpu.VMEM((ROWS_PER_TILE // CHUNK, CHUNK), jnp.int32),
        pltpu.VMEM((CHUNK, D), jnp.float32),
        pltpu.VMEM((CHUNK, D), jnp.float32),
        pltpu.VMEM((CHUNK,), jnp.float32),
        pltpu.VMEM((G_PER_TILE + L,), jnp.float32),
        pltpu.VMEM((1, L), jnp.float32),
        pltpu.SemaphoreType.DMA,
    ],
)(_sc_body)


def _reduce_body(p_ref, o_ref):
    o_ref[0, 0] = jnp.sum(p_ref[...]) * (1.0 / (B * D))


_reduce = pl.pallas_call(
    _reduce_body,
    out_shape=jax.ShapeDtypeStruct((1, 1), jnp.float32),
    out_specs=pl.BlockSpec(memory_space=pltpu.SMEM),
)


def kernel(feats, labels, cam_ids):
    partials = _sc_kernel(feats, labels, cam_ids)
    return _reduce(partials)[0, 0]


# speedup vs baseline: 4.2058x; 1.1989x over previous
import functools

import jax
import jax.numpy as jnp
from jax import lax
from jax.experimental import pallas as pl
from jax.experimental.pallas import tpu as pltpu
from jax.experimental.pallas import tpu_sc as plsc

N_LABELS = 1000
N_CAMS = 8
B = 16384
D = 128
G = N_LABELS * N_CAMS
GP = 8192
NC = 2
NS = 16
L = 16

ROWS_PER_TILE = B // NS
HALF = B // NC
LROWS = HALF // NS
CHUNK = 128
SUB = 64
G_PER_TILE = GP // NS


def _sc_body(feats_hbm, labels_hbm, cams_hbm, out_hbm,
             sums_sh, cnt_sh, labv, camv, idx_v, fbA, fbB, onesv,
             cntv, accv, semA, semB, semC, semD):
    cid = lax.axis_index("c")
    sid = lax.axis_index("s")
    bufs = (fbA, fbB)
    lsems = (semA, semB)
    wsems = (semC, semD)

    def _z(r, _):
        for c in range(D // L):
            fbA[r, pl.ds(c * L, L)] = jnp.zeros((L,), jnp.float32)
        return 0

    lax.fori_loop(0, CHUNK, _z, 0)
    for q in range(CHUNK // L):
        onesv[pl.ds(q * L, L)] = jnp.ones((L,), jnp.float32)

    def _z2(r, _):
        cntv[pl.ds(r * L, L)] = jnp.zeros((L,), jnp.float32)
        return 0

    lax.fori_loop(0, G_PER_TILE // L, _z2, 0)
    gbase = sid * G_PER_TILE
    for k in range(G_PER_TILE // CHUNK):
        pltpu.sync_copy(fbA, sums_sh.at[pl.ds(gbase + k * CHUNK, CHUNK)])
    pltpu.sync_copy(cntv.at[pl.ds(0, G_PER_TILE)],
                    cnt_sh.at[pl.ds(gbase, G_PER_TILE)])

    for h in (0, 1):
        off = ((cid + h) % 2) * HALF + sid * LROWS
        pltpu.sync_copy(labels_hbm.at[pl.ds(off, LROWS)],
                        labv.at[pl.ds(h * LROWS, LROWS)])
        pltpu.sync_copy(cams_hbm.at[pl.ds(off, LROWS)],
                        camv.at[pl.ds(h * LROWS, LROWS)])

    for q in range(ROWS_PER_TILE // L):
        g = labv[pl.ds(q * L, L)] * N_CAMS + camv[pl.ds(q * L, L)]
        idx_v[q // (CHUNK // L), pl.ds((q % (CHUNK // L)) * L, L)] = g

    plsc.subcore_barrier()

    def _p1_off(r):
        h = r // (LROWS // CHUNK)
        return (((cid + h) % 2) * HALF + sid * LROWS
                + (r % (LROWS // CHUNK)) * CHUNK)

    n1 = ROWS_PER_TILE // CHUNK
    cp = pltpu.async_copy(feats_hbm.at[pl.ds(_p1_off(0), CHUNK)], fbA, semA)
    for r in range(n1):
        cp.wait()
        if r + 1 < n1:
            cp = pltpu.async_copy(
                feats_hbm.at[pl.ds(_p1_off(r + 1), CHUNK)],
                bufs[(r + 1) % 2], lsems[(r + 1) % 2])
        pltpu.sync_copy(bufs[r % 2], sums_sh.at[idx_v.at[r]], add=True)
        pltpu.sync_copy(onesv, cnt_sh.at[idx_v.at[r]], add=True)

    plsc.subcore_barrier()

    pltpu.sync_copy(cnt_sh.at[pl.ds(gbase, G_PER_TILE)],
                    cntv.at[pl.ds(0, G_PER_TILE)])
    n2 = G_PER_TILE // CHUNK
    ld = pltpu.async_copy(sums_sh.at[pl.ds(gbase, CHUNK)], fbA, semA)
    wb = [None, None]
    for k in range(n2):
        ld.wait()
        if k + 1 < n2:
            if wb[(k + 1) % 2] is not None:
                wb[(k + 1) % 2].wait()
            ld = pltpu.async_copy(
                sums_sh.at[pl.ds(gbase + (k + 1) * CHUNK, CHUNK)],
                bufs[(k + 1) % 2], lsems[(k + 1) % 2])
        buf = bufs[k % 2]

        def _norm(r, _):
            cv = 1.0 / jnp.maximum(cntv[pl.ds(k * CHUNK + r, L)], 1.0)
            inv = jnp.full((L,), cv[0], jnp.float32)
            for c in range(D // L):
                buf[r, pl.ds(c * L, L)] = buf[r, pl.ds(c * L, L)] * inv
            return 0

        lax.fori_loop(0, CHUNK, _norm, 0)
        wb[k % 2] = pltpu.async_copy(
            buf, sums_sh.at[pl.ds(gbase + k * CHUNK, CHUNK)], wsems[k % 2])
    wb[0].wait()
    wb[1].wait()

    plsc.subcore_barrier()

    n3 = LROWS // SUB
    fh = (fbA.at[pl.ds(0, SUB)], fbA.at[pl.ds(SUB, SUB)])
    ch = (fbB.at[pl.ds(0, SUB)], fbB.at[pl.ds(SUB, SUB)])

    def _p3_start(s, fsem, gsem):
        off = cid * HALF + sid * LROWS + s * SUB
        cpf = pltpu.async_copy(feats_hbm.at[pl.ds(off, SUB)], fh[s % 2], fsem)
        idx = idx_v.at[s // 2, pl.ds((s % 2) * SUB, SUB)]
        cpg = pltpu.async_copy(sums_sh.at[idx], ch[s % 2], gsem)
        return cpf, cpg

    acc = jnp.zeros((L,), jnp.float32)
    inflight = _p3_start(0, semA, semC)
    for s in range(n3):
        inflight[0].wait()
        inflight[1].wait()
        if s + 1 < n3:
            inflight = _p3_start(s + 1, lsems[(s + 1) % 2],
                                 wsems[(s + 1) % 2])
        fbuf = fh[s % 2]
        cbuf = ch[s % 2]

        def _loss(r2, a):
            for c in range(D // L):
                f = fbuf[r2, pl.ds(c * L, L)]
                t = cbuf[r2, pl.ds(c * L, L)]
                d = f - t
                ad = jnp.abs(d)
                a = a + jnp.where(ad < 1.0, 0.5 * d * d, ad - 0.5)
            return a

        acc = lax.fori_loop(0, SUB, _loss, acc)

    accv[pl.ds(0, 1), pl.ds(0, L)] = acc.reshape(1, L)
    wid = cid * NS + sid
    pltpu.sync_copy(accv, out_hbm.at[pl.ds(wid, 1)])


_sc_kernel = functools.partial(
    pl.kernel,
    out_type=jax.ShapeDtypeStruct((NC * NS, L), jnp.float32),
    mesh=plsc.VectorSubcoreMesh(core_axis_name="c", subcore_axis_name="s"),
    scratch_types=[
        pltpu.VMEM_SHARED((GP, D), jnp.float32),
        pltpu.VMEM_SHARED((GP,), jnp.float32),
        pltpu.VMEM((ROWS_PER_TILE,), jnp.int32),
        pltpu.VMEM((ROWS_PER_TILE,), jnp.int32),
        pltpu.VMEM((ROWS_PER_TILE // CHUNK, CHUNK), jnp.int32),
        pltpu.VMEM((CHUNK, D), jnp.float32),
        pltpu.VMEM((CHUNK, D), jnp.float32),
        pltpu.VMEM((CHUNK,), jnp.float32),
        pltpu.VMEM((G_PER_TILE + L,), jnp.float32),
        pltpu.VMEM((1, L), jnp.float32),
        pltpu.SemaphoreType.DMA,
        pltpu.SemaphoreType.DMA,
        pltpu.SemaphoreType.DMA,
        pltpu.SemaphoreType.DMA,
    ],
)(_sc_body)


def _reduce_body(p_ref, o_ref):
    o_ref[0, 0] = jnp.sum(p_ref[...]) * (1.0 / (B * D))


_reduce = pl.pallas_call(
    _reduce_body,
    out_shape=jax.ShapeDtypeStruct((1, 1), jnp.float32),
    out_specs=pl.BlockSpec(memory_space=pltpu.SMEM),
)


def kernel(feats, labels, cam_ids):
    partials = _sc_kernel(feats, labels, cam_ids)
    return _reduce(partials)[0, 0]


# speedup vs baseline: 4.6556x; 1.1069x over previous
import functools

import jax
import jax.numpy as jnp
from jax import lax
from jax.experimental import pallas as pl
from jax.experimental.pallas import tpu as pltpu
from jax.experimental.pallas import tpu_sc as plsc

N_LABELS = 1000
N_CAMS = 8
B = 16384
D = 128
G = N_LABELS * N_CAMS
GP = 8192
NC = 2
NS = 16
L = 16

ROWS_PER_TILE = B // NS
HALF = B // NC
LROWS = HALF // NS
CHUNK = 128
SUB = 64
G_PER_TILE = GP // NS


def _sc_body(feats_hbm, labels_hbm, cams_hbm, out_hbm,
             sums_sh, cnt_sh, labv, camv, idx_v, fbA, fbB, onesv,
             cntv, accv, semA, semB, semC, semD):
    cid = lax.axis_index("c")
    sid = lax.axis_index("s")
    bufs = (fbA, fbB)
    lsems = (semA, semB)
    wsems = (semC, semD)

    def _z(r, _):
        for c in range(D // L):
            fbA[r, pl.ds(c * L, L)] = jnp.zeros((L,), jnp.float32)
        return 0

    lax.fori_loop(0, CHUNK, _z, 0)
    for q in range(CHUNK // L):
        onesv[pl.ds(q * L, L)] = jnp.ones((L,), jnp.float32)

    def _z2(r, _):
        cntv[pl.ds(r * L, L)] = jnp.zeros((L,), jnp.float32)
        return 0

    lax.fori_loop(0, G_PER_TILE // L, _z2, 0)
    gbase = sid * G_PER_TILE
    for k in range(G_PER_TILE // CHUNK):
        pltpu.sync_copy(fbA, sums_sh.at[pl.ds(gbase + k * CHUNK, CHUNK)])
    pltpu.sync_copy(cntv.at[pl.ds(0, G_PER_TILE)],
                    cnt_sh.at[pl.ds(gbase, G_PER_TILE)])

    for h in (0, 1):
        off = ((cid + h) % 2) * HALF + sid * LROWS
        pltpu.sync_copy(labels_hbm.at[pl.ds(off, LROWS)],
                        labv.at[pl.ds(h * LROWS, LROWS)])
        pltpu.sync_copy(cams_hbm.at[pl.ds(off, LROWS)],
                        camv.at[pl.ds(h * LROWS, LROWS)])

    for q in range(ROWS_PER_TILE // L):
        g = labv[pl.ds(q * L, L)] * N_CAMS + camv[pl.ds(q * L, L)]
        idx_v[q // (CHUNK // L), pl.ds((q % (CHUNK // L)) * L, L)] = g

    plsc.subcore_barrier()

    def _p1_off(r):
        h = r // (LROWS // CHUNK)
        return (((cid + h) % 2) * HALF + sid * LROWS
                + (r % (LROWS // CHUNK)) * CHUNK)

    n1 = ROWS_PER_TILE // CHUNK
    cp = pltpu.async_copy(feats_hbm.at[pl.ds(_p1_off(0), CHUNK)], fbA, semA)
    for r in range(n1):
        cp.wait()
        if r + 1 < n1:
            cp = pltpu.async_copy(
                feats_hbm.at[pl.ds(_p1_off(r + 1), CHUNK)],
                bufs[(r + 1) % 2], lsems[(r + 1) % 2])
        pltpu.sync_copy(bufs[r % 2], sums_sh.at[idx_v.at[r]], add=True)
        pltpu.sync_copy(onesv, cnt_sh.at[idx_v.at[r]], add=True)

    plsc.subcore_barrier()

    pltpu.sync_copy(cnt_sh.at[pl.ds(gbase, G_PER_TILE)],
                    cntv.at[pl.ds(0, G_PER_TILE)])
    n2 = G_PER_TILE // CHUNK
    ld = pltpu.async_copy(sums_sh.at[pl.ds(gbase, CHUNK)], fbA, semA)
    wb = [None, None]
    for k in range(n2):
        ld.wait()
        if k + 1 < n2:
            if wb[(k + 1) % 2] is not None:
                wb[(k + 1) % 2].wait()
            ld = pltpu.async_copy(
                sums_sh.at[pl.ds(gbase + (k + 1) * CHUNK, CHUNK)],
                bufs[(k + 1) % 2], lsems[(k + 1) % 2])
        buf = bufs[k % 2]

        @plsc.parallel_loop(0, CHUNK, unroll=2)
        def _norm(r):
            cv = 1.0 / jnp.maximum(cntv[pl.ds(k * CHUNK + r, L)], 1.0)
            inv = jnp.full((L,), cv[0], jnp.float32)
            for c in range(D // L):
                buf[r, pl.ds(c * L, L)] = buf[r, pl.ds(c * L, L)] * inv
        wb[k % 2] = pltpu.async_copy(
            buf, sums_sh.at[pl.ds(gbase + k * CHUNK, CHUNK)], wsems[k % 2])
    wb[0].wait()
    wb[1].wait()

    plsc.subcore_barrier()

    n3 = LROWS // SUB
    fh = (fbA.at[pl.ds(0, SUB)], fbA.at[pl.ds(SUB, SUB)])
    ch = (fbB.at[pl.ds(0, SUB)], fbB.at[pl.ds(SUB, SUB)])

    def _p3_start(s, fsem, gsem):
        off = cid * HALF + sid * LROWS + s * SUB
        cpf = pltpu.async_copy(feats_hbm.at[pl.ds(off, SUB)], fh[s % 2], fsem)
        idx = idx_v.at[s // 2, pl.ds((s % 2) * SUB, SUB)]
        cpg = pltpu.async_copy(sums_sh.at[idx], ch[s % 2], gsem)
        return cpf, cpg

    accs = tuple(jnp.zeros((L,), jnp.float32) for _ in range(D // L))
    inflight = _p3_start(0, semA, semC)
    for s in range(n3):
        inflight[0].wait()
        inflight[1].wait()
        if s + 1 < n3:
            inflight = _p3_start(s + 1, lsems[(s + 1) % 2],
                                 wsems[(s + 1) % 2])
        fbuf = fh[s % 2]
        cbuf = ch[s % 2]

        @plsc.parallel_loop(0, SUB, unroll=2, carry=accs)
        def _loss(r2, a):
            out = []
            for c in range(D // L):
                f = fbuf[r2, pl.ds(c * L, L)]
                t = cbuf[r2, pl.ds(c * L, L)]
                d = f - t
                ad = jnp.abs(d)
                out.append(a[c] + jnp.where(ad < 1.0, 0.5 * d * d, ad - 0.5))
            return tuple(out)

        accs = _loss

    acc = accs[0]
    for c in range(1, D // L):
        acc = acc + accs[c]
    accv[pl.ds(0, 1), pl.ds(0, L)] = acc.reshape(1, L)
    wid = cid * NS + sid
    pltpu.sync_copy(accv, out_hbm.at[pl.ds(wid, 1)])


_sc_kernel = functools.partial(
    pl.kernel,
    out_type=jax.ShapeDtypeStruct((NC * NS, L), jnp.float32),
    mesh=plsc.VectorSubcoreMesh(core_axis_name="c", subcore_axis_name="s"),
    scratch_types=[
        pltpu.VMEM_SHARED((GP, D), jnp.float32),
        pltpu.VMEM_SHARED((GP,), jnp.float32),
        pltpu.VMEM((ROWS_PER_TILE,), jnp.int32),
        pltpu.VMEM((ROWS_PER_TILE,), jnp.int32),
        pltpu.VMEM((ROWS_PER_TILE // CHUNK, CHUNK), jnp.int32),
        pltpu.VMEM((CHUNK, D), jnp.float32),
        pltpu.VMEM((CHUNK, D), jnp.float32),
        pltpu.VMEM((CHUNK,), jnp.float32),
        pltpu.VMEM((G_PER_TILE + L,), jnp.float32),
        pltpu.VMEM((1, L), jnp.float32),
        pltpu.SemaphoreType.DMA,
        pltpu.SemaphoreType.DMA,
        pltpu.SemaphoreType.DMA,
        pltpu.SemaphoreType.DMA,
    ],
)(_sc_body)


def _reduce_body(p_ref, o_ref):
    o_ref[0, 0] = jnp.sum(p_ref[...]) * (1.0 / (B * D))


_reduce = pl.pallas_call(
    _reduce_body,
    out_shape=jax.ShapeDtypeStruct((1, 1), jnp.float32),
    out_specs=pl.BlockSpec(memory_space=pltpu.SMEM),
)


def kernel(feats, labels, cam_ids):
    partials = _sc_kernel(feats, labels, cam_ids)
    return _reduce(partials)[0, 0]


# speedup vs baseline: 4.8849x; 1.0493x over previous
import functools

import jax
import jax.numpy as jnp
from jax import lax
from jax.experimental import pallas as pl
from jax.experimental.pallas import tpu as pltpu
from jax.experimental.pallas import tpu_sc as plsc

N_LABELS = 1000
N_CAMS = 8
B = 16384
D = 128
G = N_LABELS * N_CAMS
GP = 8192
NC = 2
NS = 16
L = 16

ROWS_PER_TILE = B // NS
HALF = B // NC
LROWS = HALF // NS
CHUNK = 128
SUB = 64
G_PER_TILE = GP // NS


def _sc_body(feats_hbm, labels_hbm, cams_hbm, out_hbm,
             sums_sh, cnt_sh, labv, camv, idx_v, fbA, fbB, onesv,
             cntv, accv, semA, semB, semC, semD, semE):
    cid = lax.axis_index("c")
    sid = lax.axis_index("s")
    bufs = (fbA, fbB)
    lsems = (semA, semB)
    wsems = (semC, semD)

    def _z(r, _):
        for c in range(D // L):
            fbA[r, pl.ds(c * L, L)] = jnp.zeros((L,), jnp.float32)
        return 0

    lab_cps = []
    for h in (0, 1):
        off = ((cid + h) % 2) * HALF + sid * LROWS
        lab_cps.append(pltpu.async_copy(
            labels_hbm.at[pl.ds(off, LROWS)],
            labv.at[pl.ds(h * LROWS, LROWS)], semE))
        lab_cps.append(pltpu.async_copy(
            cams_hbm.at[pl.ds(off, LROWS)],
            camv.at[pl.ds(h * LROWS, LROWS)], semE))

    lax.fori_loop(0, CHUNK, _z, 0)
    for q in range(CHUNK // L):
        onesv[pl.ds(q * L, L)] = jnp.ones((L,), jnp.float32)

    def _z2(r, _):
        cntv[pl.ds(r * L, L)] = jnp.zeros((L,), jnp.float32)
        return 0

    lax.fori_loop(0, G_PER_TILE // L, _z2, 0)
    gbase = sid * G_PER_TILE
    zero_cps = [
        pltpu.async_copy(fbA, sums_sh.at[pl.ds(gbase + k * CHUNK, CHUNK)],
                         semA)
        for k in range(G_PER_TILE // CHUNK)
    ]
    zero_cps.append(pltpu.async_copy(
        cntv.at[pl.ds(0, G_PER_TILE)],
        cnt_sh.at[pl.ds(gbase, G_PER_TILE)], semB))

    for cp0 in lab_cps:
        cp0.wait()

    for q in range(ROWS_PER_TILE // L):
        g = labv[pl.ds(q * L, L)] * N_CAMS + camv[pl.ds(q * L, L)]
        idx_v[q // (CHUNK // L), pl.ds((q % (CHUNK // L)) * L, L)] = g

    for cp0 in zero_cps:
        cp0.wait()
    plsc.subcore_barrier()

    def _p1_off(r):
        h = r // (LROWS // CHUNK)
        return (((cid + h) % 2) * HALF + sid * LROWS
                + (r % (LROWS // CHUNK)) * CHUNK)

    n1 = ROWS_PER_TILE // CHUNK
    scat = [None, None]
    cnt_cps = []
    cp = pltpu.async_copy(feats_hbm.at[pl.ds(_p1_off(0), CHUNK)], fbA, semA)
    for r in range(n1):
        cp.wait()
        if r + 1 < n1:
            if scat[(r + 1) % 2] is not None:
                scat[(r + 1) % 2].wait()
            cp = pltpu.async_copy(
                feats_hbm.at[pl.ds(_p1_off(r + 1), CHUNK)],
                bufs[(r + 1) % 2], lsems[(r + 1) % 2])
        scat[r % 2] = pltpu.async_copy(
            bufs[r % 2], sums_sh.at[idx_v.at[r]], wsems[r % 2], add=True)
        cnt_cps.append(pltpu.async_copy(
            onesv, cnt_sh.at[idx_v.at[r]], semE, add=True))
    scat[0].wait()
    scat[1].wait()
    for cp1 in cnt_cps:
        cp1.wait()

    plsc.subcore_barrier()

    pltpu.sync_copy(cnt_sh.at[pl.ds(gbase, G_PER_TILE)],
                    cntv.at[pl.ds(0, G_PER_TILE)])
    n2 = G_PER_TILE // CHUNK
    ld = pltpu.async_copy(sums_sh.at[pl.ds(gbase, CHUNK)], fbA, semA)
    wb = [None, None]
    for k in range(n2):
        ld.wait()
        if k + 1 < n2:
            if wb[(k + 1) % 2] is not None:
                wb[(k + 1) % 2].wait()
            ld = pltpu.async_copy(
                sums_sh.at[pl.ds(gbase + (k + 1) * CHUNK, CHUNK)],
                bufs[(k + 1) % 2], lsems[(k + 1) % 2])
        buf = bufs[k % 2]

        @plsc.parallel_loop(0, CHUNK, unroll=2)
        def _norm(r):
            cv = 1.0 / jnp.maximum(cntv[pl.ds(k * CHUNK + r, L)], 1.0)
            inv = jnp.full((L,), cv[0], jnp.float32)
            for c in range(D // L):
                buf[r, pl.ds(c * L, L)] = buf[r, pl.ds(c * L, L)] * inv
        wb[k % 2] = pltpu.async_copy(
            buf, sums_sh.at[pl.ds(gbase + k * CHUNK, CHUNK)], wsems[k % 2])
    wb[0].wait()
    wb[1].wait()

    plsc.subcore_barrier()

    n3 = LROWS // SUB
    fh = (fbA.at[pl.ds(0, SUB)], fbA.at[pl.ds(SUB, SUB)])
    ch = (fbB.at[pl.ds(0, SUB)], fbB.at[pl.ds(SUB, SUB)])

    def _p3_start(s, fsem, gsem):
        off = cid * HALF + sid * LROWS + s * SUB
        cpf = pltpu.async_copy(feats_hbm.at[pl.ds(off, SUB)], fh[s % 2], fsem)
        idx = idx_v.at[s // 2, pl.ds((s % 2) * SUB, SUB)]
        cpg = pltpu.async_copy(sums_sh.at[idx], ch[s % 2], gsem)
        return cpf, cpg

    accs = tuple(jnp.zeros((L,), jnp.float32) for _ in range(D // L))
    inflight = _p3_start(0, semA, semC)
    for s in range(n3):
        inflight[0].wait()
        inflight[1].wait()
        if s + 1 < n3:
            inflight = _p3_start(s + 1, lsems[(s + 1) % 2],
                                 wsems[(s + 1) % 2])
        fbuf = fh[s % 2]
        cbuf = ch[s % 2]

        @plsc.parallel_loop(0, SUB, unroll=2, carry=accs)
        def _loss(r2, a):
            out = []
            for c in range(D // L):
                f = fbuf[r2, pl.ds(c * L, L)]
                t = cbuf[r2, pl.ds(c * L, L)]
                d = f - t
                ad = jnp.abs(d)
                out.append(a[c] + jnp.where(ad < 1.0, 0.5 * d * d, ad - 0.5))
            return tuple(out)

        accs = _loss

    acc = accs[0]
    for c in range(1, D // L):
        acc = acc + accs[c]
    accv[pl.ds(0, 1), pl.ds(0, L)] = acc.reshape(1, L)
    wid = cid * NS + sid
    pltpu.sync_copy(accv, out_hbm.at[pl.ds(wid, 1)])


_sc_kernel = functools.partial(
    pl.kernel,
    out_type=jax.ShapeDtypeStruct((NC * NS, L), jnp.float32),
    mesh=plsc.VectorSubcoreMesh(core_axis_name="c", subcore_axis_name="s"),
    scratch_types=[
        pltpu.VMEM_SHARED((GP, D), jnp.float32),
        pltpu.VMEM_SHARED((GP,), jnp.float32),
        pltpu.VMEM((ROWS_PER_TILE,), jnp.int32),
        pltpu.VMEM((ROWS_PER_TILE,), jnp.int32),
        pltpu.VMEM((ROWS_PER_TILE // CHUNK, CHUNK), jnp.int32),
        pltpu.VMEM((CHUNK, D), jnp.float32),
        pltpu.VMEM((CHUNK, D), jnp.float32),
        pltpu.VMEM((CHUNK,), jnp.float32),
        pltpu.VMEM((G_PER_TILE + L,), jnp.float32),
        pltpu.VMEM((1, L), jnp.float32),
        pltpu.SemaphoreType.DMA,
        pltpu.SemaphoreType.DMA,
        pltpu.SemaphoreType.DMA,
        pltpu.SemaphoreType.DMA,
        pltpu.SemaphoreType.DMA,
    ],
)(_sc_body)


def _reduce_body(p_ref, o_ref):
    o_ref[0, 0] = jnp.sum(p_ref[...]) * (1.0 / (B * D))


_reduce = pl.pallas_call(
    _reduce_body,
    out_shape=jax.ShapeDtypeStruct((1, 1), jnp.float32),
    out_specs=pl.BlockSpec(memory_space=pltpu.SMEM),
)


def kernel(feats, labels, cam_ids):
    partials = _sc_kernel(feats, labels, cam_ids)
    return _reduce(partials)[0, 0]


# speedup vs baseline: 5.5270x; 1.1314x over previous
import functools

import jax
import jax.numpy as jnp
from jax import lax
from jax.experimental import pallas as pl
from jax.experimental.pallas import tpu as pltpu
from jax.experimental.pallas import tpu_sc as plsc

N_LABELS = 1000
N_CAMS = 8
B = 16384
D = 128
G = N_LABELS * N_CAMS
GP = 8192
NC = 2
NS = 16
L = 16

ROWS_PER_TILE = B // NS
HALF = B // NC
LROWS = HALF // NS
CHUNK = 128
SUB = 64
G_PER_TILE = GP // NS


def _sc_body(feats_hbm, labels_hbm, cams_hbm, out_hbm,
             sums_sh, cnt_sh, labv, camv, idx_v, fbA, fbB, onesv,
             cntv, accv, semA, semB, semC, semD, semE):
    cid = lax.axis_index("c")
    sid = lax.axis_index("s")
    bufs = (fbA, fbB)
    lsems = (semA, semB)
    wsems = (semC, semD)

    def _z(r, _):
        for c in range(D // L):
            fbA[r, pl.ds(c * L, L)] = jnp.zeros((L,), jnp.float32)
        return 0

    lab_cps = []
    for h in (0, 1):
        off = ((cid + h) % 2) * HALF + sid * LROWS
        lab_cps.append(pltpu.async_copy(
            labels_hbm.at[pl.ds(off, LROWS)],
            labv.at[pl.ds(h * LROWS, LROWS)], semE))
        lab_cps.append(pltpu.async_copy(
            cams_hbm.at[pl.ds(off, LROWS)],
            camv.at[pl.ds(h * LROWS, LROWS)], semE))

    lax.fori_loop(0, CHUNK, _z, 0)
    for q in range(CHUNK // L):
        onesv[pl.ds(q * L, L)] = jnp.ones((L,), jnp.float32)

    def _z2(r, _):
        cntv[pl.ds(r * L, L)] = jnp.zeros((L,), jnp.float32)
        return 0

    lax.fori_loop(0, G_PER_TILE // L, _z2, 0)
    gbase = sid * G_PER_TILE
    zero_cps = [
        pltpu.async_copy(fbA, sums_sh.at[pl.ds(gbase + k * CHUNK, CHUNK)],
                         semA)
        for k in range(G_PER_TILE // CHUNK)
    ]
    zero_cps.append(pltpu.async_copy(
        cntv.at[pl.ds(0, G_PER_TILE)],
        cnt_sh.at[pl.ds(gbase, G_PER_TILE)], semB))

    for cp0 in lab_cps:
        cp0.wait()

    for q in range(ROWS_PER_TILE // L):
        g = labv[pl.ds(q * L, L)] * N_CAMS + camv[pl.ds(q * L, L)]
        idx_v[q // (CHUNK // L), pl.ds((q % (CHUNK // L)) * L, L)] = g

    for cp0 in zero_cps:
        cp0.wait()
    plsc.subcore_barrier()

    def _p1_off(r):
        h = r // (LROWS // CHUNK)
        return (((cid + h) % 2) * HALF + sid * LROWS
                + (r % (LROWS // CHUNK)) * CHUNK)

    n1 = ROWS_PER_TILE // CHUNK
    scat = [None, None]
    cnt_cps = []
    cp = pltpu.async_copy(feats_hbm.at[pl.ds(_p1_off(0), CHUNK)], fbA, semA)
    for r in range(n1):
        cp.wait()
        if r + 1 < n1:
            if scat[(r + 1) % 2] is not None:
                scat[(r + 1) % 2].wait()
            cp = pltpu.async_copy(
                feats_hbm.at[pl.ds(_p1_off(r + 1), CHUNK)],
                bufs[(r + 1) % 2], lsems[(r + 1) % 2])
        scat[r % 2] = pltpu.async_copy(
            bufs[r % 2], sums_sh.at[idx_v.at[r]], wsems[r % 2], add=True)
        cnt_cps.append(pltpu.async_copy(
            onesv, cnt_sh.at[idx_v.at[r]], semE, add=True))
    scat[0].wait()
    scat[1].wait()
    for cp1 in cnt_cps:
        cp1.wait()

    plsc.subcore_barrier()

    n3 = LROWS // SUB
    fh = (fbA.at[pl.ds(0, SUB)], fbA.at[pl.ds(SUB, SUB)])
    ch = (fbB.at[pl.ds(0, SUB)], fbB.at[pl.ds(SUB, SUB)])

    def _p3_start(s, fsem, gsem):
        off = cid * HALF + sid * LROWS + s * SUB
        cpf = pltpu.async_copy(feats_hbm.at[pl.ds(off, SUB)], fh[s % 2], fsem)
        idx = idx_v.at[s // 2, pl.ds((s % 2) * SUB, SUB)]
        cpg = pltpu.async_copy(sums_sh.at[idx], ch[s % 2], gsem)
        cpc = pltpu.async_copy(cnt_sh.at[idx],
                               cntv.at[pl.ds((s % 2) * 80, SUB)], gsem)
        return cpf, cpg, cpc

    accs = tuple(jnp.zeros((L,), jnp.float32) for _ in range(D // L))
    inflight = _p3_start(0, semA, semC)
    for s in range(n3):
        inflight[0].wait()
        inflight[1].wait()
        inflight[2].wait()
        if s + 1 < n3:
            inflight = _p3_start(s + 1, lsems[(s + 1) % 2],
                                 wsems[(s + 1) % 2])
        fbuf = fh[s % 2]
        cbuf = ch[s % 2]

        cbase = (s % 2) * 80

        @plsc.parallel_loop(0, SUB, unroll=2, carry=accs)
        def _loss(r2, a):
            iv = 1.0 / jnp.maximum(cntv[pl.ds(cbase + r2, L)], 1.0)
            inv = jnp.full((L,), iv[0], jnp.float32)
            out = []
            for c in range(D // L):
                f = fbuf[r2, pl.ds(c * L, L)]
                t = cbuf[r2, pl.ds(c * L, L)] * inv
                d = f - t
                ad = jnp.abs(d)
                out.append(a[c] + jnp.where(ad < 1.0, 0.5 * d * d, ad - 0.5))
            return tuple(out)

        accs = _loss

    acc = accs[0]
    for c in range(1, D // L):
        acc = acc + accs[c]
    accv[pl.ds(0, 1), pl.ds(0, L)] = acc.reshape(1, L)
    wid = cid * NS + sid
    pltpu.sync_copy(accv, out_hbm.at[pl.ds(wid, 1)])


_sc_kernel = functools.partial(
    pl.kernel,
    out_type=jax.ShapeDtypeStruct((NC * NS, L), jnp.float32),
    mesh=plsc.VectorSubcoreMesh(core_axis_name="c", subcore_axis_name="s"),
    scratch_types=[
        pltpu.VMEM_SHARED((GP, D), jnp.float32),
        pltpu.VMEM_SHARED((GP,), jnp.float32),
        pltpu.VMEM((ROWS_PER_TILE,), jnp.int32),
        pltpu.VMEM((ROWS_PER_TILE,), jnp.int32),
        pltpu.VMEM((ROWS_PER_TILE // CHUNK, CHUNK), jnp.int32),
        pltpu.VMEM((CHUNK, D), jnp.float32),
        pltpu.VMEM((CHUNK, D), jnp.float32),
        pltpu.VMEM((CHUNK,), jnp.float32),
        pltpu.VMEM((G_PER_TILE + L,), jnp.float32),
        pltpu.VMEM((1, L), jnp.float32),
        pltpu.SemaphoreType.DMA,
        pltpu.SemaphoreType.DMA,
        pltpu.SemaphoreType.DMA,
        pltpu.SemaphoreType.DMA,
        pltpu.SemaphoreType.DMA,
    ],
)(_sc_body)


def _reduce_body(p_ref, o_ref):
    o_ref[0, 0] = jnp.sum(p_ref[...]) * (1.0 / (B * D))


_reduce = pl.pallas_call(
    _reduce_body,
    out_shape=jax.ShapeDtypeStruct((1, 1), jnp.float32),
    out_specs=pl.BlockSpec(memory_space=pltpu.SMEM),
)


def kernel(feats, labels, cam_ids):
    partials = _sc_kernel(feats, labels, cam_ids)
    return _reduce(partials)[0, 0]
